# baseline (speedup 1.0000x reference)
import functools

import jax
import jax.numpy as jnp
from jax import lax
from jax.experimental import pallas as pl
from jax.experimental.pallas import tpu as pltpu
from jax.experimental.pallas import tpu_sc as plsc

NUM_CORES = 2
NUM_SUBCORES = 16
NUM_WORKERS = NUM_CORES * NUM_SUBCORES
BLK = 128
NBUF = 5


@functools.partial(jax.jit, static_argnames=("seq", "bsz", "dim"))
def _gather_sc(idx3, table, *, seq, bsz, dim):
    bpl = bsz // BLK
    nblk = (seq * bpl) // NUM_WORKERS
    dtr = dim // 8
    mesh = plsc.VectorSubcoreMesh(
        core_axis_name="c", subcore_axis_name="s",
        num_cores=NUM_CORES, num_subcores=NUM_SUBCORES)

    @functools.partial(
        pl.kernel,
        out_type=jax.ShapeDtypeStruct((seq, dtr, bpl, 8, BLK, 1), table.dtype),
        mesh=mesh,
        compiler_params=pltpu.CompilerParams(use_tc_tiling_on_sc=False),
        scratch_types=(
            [pltpu.VMEM((nblk, BLK), jnp.int32),
             pltpu.VMEM((NBUF, BLK, dim), table.dtype)]
            + [pltpu.SemaphoreType.DMA] * (2 * NBUF)
        ),
    )
    def body(idx_hbm, table_hbm, out_hbm, idx_v, rows_v, *sems):
        gsems = sems[:NBUF]
        wsems = sems[NBUF:]
        wid = lax.axis_index("s") * NUM_CORES + lax.axis_index("c")
        pltpu.sync_copy(idx_hbm.at[wid], idx_v)

        def gather(j, bb):
            return pltpu.make_async_copy(
                table_hbm.at[idx_v.at[j]], rows_v.at[bb], gsems[bb])

        def wb_copies(j, bb):
            blk_id = wid * nblk + j
            l = blk_id // bpl
            bc = blk_id % bpl
            return [
                pltpu.make_async_copy(
                    rows_v.at[bb].at[:, pl.ds(t * 8 + s, 1)],
                    out_hbm.at[l, t, bc, s],
                    wsems[bb])
                for t in range(dtr)
                for s in range(8)
            ]

        gather(0, 0).start()

        def step(go, carry):
            for bb in range(NBUF):
                j = go * NBUF + bb
                gather(j, bb).wait()
                for cp in wb_copies(j, bb):
                    cp.start()
                bn = (bb + 1) % NBUF

                @pl.when(j + 1 < nblk)
                def _():
                    @pl.when(j - (NBUF - 1) >= 0)
                    def _():
                        for cp in wb_copies(j - (NBUF - 1), bn):
                            cp.wait()
                    gather(j + 1, bn).start()
            return carry

        lax.fori_loop(0, nblk // NBUF, step, 0)
        for bb in range(NBUF):
            j = nblk - NBUF + bb
            for cp in wb_copies(j, (j % NBUF)):
                cp.wait()

    return body(idx3, table)


def kernel(indices, table):
    bsz, seq = indices.shape
    dim = table.shape[1]
    assert bsz % BLK == 0 and dim % 8 == 0
    assert (seq * (bsz // BLK)) % (NUM_WORKERS * NBUF) == 0
    nblk = (seq * (bsz // BLK)) // NUM_WORKERS
    idx3 = indices.astype(jnp.int32).T.reshape(NUM_WORKERS, nblk, BLK)
    out = _gather_sc(idx3, table, seq=seq, bsz=bsz, dim=dim)
    out = out.reshape(seq, dim // 8, bsz // BLK, 8, BLK)
    out = out.transpose(2, 4, 0, 1, 3).reshape(bsz, seq, dim)
    return out


# speedup vs baseline: 40.1401x; 40.1401x over previous
import functools

import jax
import jax.numpy as jnp
from jax import lax
from jax.experimental import pallas as pl
from jax.experimental.pallas import tpu as pltpu
from jax.experimental.pallas import tpu_sc as plsc

NUM_CORES = 2
NUM_SUBCORES = 16
NUM_WORKERS = NUM_CORES * NUM_SUBCORES
CHUNK = 128
GS = 5
NBUF = 2


@functools.partial(jax.jit, static_argnames=("total", "dim", "nchunk"))
def _gather_sc(idx, table, *, total, dim, nchunk):
    ngrp = nchunk // GS
    grows = GS * CHUNK
    mesh = plsc.VectorSubcoreMesh(
        core_axis_name="c", subcore_axis_name="s",
        num_cores=NUM_CORES, num_subcores=NUM_SUBCORES)

    @functools.partial(
        pl.kernel,
        out_type=jax.ShapeDtypeStruct((total, dim), table.dtype),
        mesh=mesh,
        compiler_params=pltpu.CompilerParams(use_tc_tiling_on_sc=False),
        scratch_types=[
            pltpu.VMEM((nchunk, CHUNK), jnp.int32),
            pltpu.VMEM((NBUF, grows, dim), table.dtype),
            pltpu.SemaphoreType.DMA,
            pltpu.SemaphoreType.DMA,
            pltpu.SemaphoreType.DMA,
            pltpu.SemaphoreType.DMA,
        ],
    )
    def body(idx_hbm, table_hbm, out_hbm, idx_v, rows_v, g0, g1, w0, w1):
        gsems = (g0, g1)
        wsems = (w0, w1)
        wid = lax.axis_index("s") * NUM_CORES + lax.axis_index("c")
        base = wid * (nchunk * CHUNK)
        pltpu.sync_copy(idx_hbm.at[wid], idx_v)

        def fire(g, gb):
            for c in range(GS):
                pltpu.async_copy(
                    table_hbm.at[idx_v.at[g * GS + c]],
                    rows_v.at[gb].at[pl.ds(c * CHUNK, CHUNK)],
                    gsems[gb])

        def drain(g, gb):
            for c in range(GS):
                pltpu.make_async_copy(
                    table_hbm.at[idx_v.at[g * GS + c]],
                    rows_v.at[gb].at[pl.ds(c * CHUNK, CHUNK)],
                    gsems[gb]).wait()

        def wb(g, gb):
            return pltpu.make_async_copy(
                rows_v.at[gb], out_hbm.at[pl.ds(base + g * grows, grows)],
                wsems[gb])

        fire(0, 0)

        def step(go, carry):
            for gg in range(NBUF):
                g = go * NBUF + gg
                nxt = g + 1
                @pl.when(nxt < ngrp)
                def _():
                    @pl.when(g >= 1)
                    def _():
                        wb(g - 1, 1 - gg).wait()
                    fire(nxt, 1 - gg)

                drain(g, gg)
                wb(g, gg).start()
            return carry

        lax.fori_loop(0, ngrp // NBUF, step, 0)
        for gg in range(NBUF):
            wb(ngrp - NBUF + gg, gg).wait()

    return body(idx, table)


def kernel(indices, table):
    bsz, seq = indices.shape
    dim = table.shape[1]
    total = bsz * seq
    assert total % (NUM_WORKERS * CHUNK * GS * NBUF) == 0
    nchunk = total // (NUM_WORKERS * CHUNK)
    idx = indices.astype(jnp.int32).T.reshape(NUM_WORKERS, nchunk, CHUNK)
    out = _gather_sc(idx, table, total=total, dim=dim, nchunk=nchunk)
    return out.reshape(seq, bsz, dim).transpose(1, 0, 2)
